# baseline (speedup 1.0000x reference)
import functools

import jax
import jax.numpy as jnp
from jax.experimental import pallas as pl
from jax.experimental.pallas import tpu as pltpu

TOPK = 64
TM = 256
TL = 1024
CHUNK = 128


def _enc_body(x_ref, wenc_ref, benc_ref, bdec_ref, pre_ref):
    xc = x_ref[...] - bdec_ref[...]
    acc = jnp.dot(xc, wenc_ref[...], preferred_element_type=jnp.float32)
    pre_ref[...] = jnp.maximum(acc + benc_ref[...], 0.0)


def _encode(x, W_enc, b_enc, b_dec):
    N, D = x.shape
    L = W_enc.shape[1]
    grid = (N // TM, L // TL)
    pre = pl.pallas_call(
        _enc_body,
        grid=grid,
        in_specs=[
            pl.BlockSpec((TM, D), lambda i, j: (i, 0)),
            pl.BlockSpec((D, TL), lambda i, j: (0, j)),
            pl.BlockSpec((1, TL), lambda i, j: (0, j)),
            pl.BlockSpec((1, D), lambda i, j: (0, 0)),
        ],
        out_specs=pl.BlockSpec((TM, TL), lambda i, j: (i, j)),
        out_shape=jax.ShapeDtypeStruct((N, L), jnp.float32),
    )(x, W_enc, b_enc.reshape(1, L), b_dec.reshape(1, D))
    return pre


def kernel(x, W_enc, b_enc, W_dec, b_dec):
    pre_acts = _encode(x, W_enc, b_enc, b_dec)
    top_acts, top_indices = jax.lax.top_k(pre_acts, TOPK)
    dec_rows = jnp.take(W_dec, top_indices, axis=0)
    sae_out = jnp.einsum('nk,nkd->nd', top_acts, dec_rows) + b_dec
    e = x - sae_out
    total_variance = jnp.sum((x - jnp.mean(x, axis=0)) ** 2)
    l2_loss = jnp.sum(e ** 2)
    fvu = l2_loss / total_variance
    return sae_out, top_acts, top_indices, fvu


# speedup vs baseline: 6.1895x; 6.1895x over previous
import functools

import jax
import jax.numpy as jnp
from jax import lax
from jax.experimental import pallas as pl
from jax.experimental.pallas import tpu as pltpu
from jax.experimental.pallas import tpu_sc as plsc

TOPK = 64
CHUNK = 128
NCAND = 80
TM = 256
TL = 1024
TM2 = 128


def _enc_body(x_ref, wenc_ref, benc_ref, bdec_ref, pre_ref):
    xc = x_ref[...] - bdec_ref[...]
    acc = jnp.dot(xc, wenc_ref[...], preferred_element_type=jnp.float32)
    pre_ref[...] = jnp.maximum(acc + benc_ref[...], 0.0)


def _encode(x, W_enc, b_enc, b_dec):
    N, D = x.shape
    L = W_enc.shape[1]
    return pl.pallas_call(
        _enc_body,
        grid=(N // TM, L // TL),
        in_specs=[
            pl.BlockSpec((TM, D), lambda i, j: (i, 0)),
            pl.BlockSpec((D, TL), lambda i, j: (0, j)),
            pl.BlockSpec((1, TL), lambda i, j: (0, j)),
            pl.BlockSpec((1, D), lambda i, j: (0, 0)),
        ],
        out_specs=pl.BlockSpec((TM, TL), lambda i, j: (i, j)),
        out_shape=jax.ShapeDtypeStruct((N, L), jnp.float32),
    )(x, W_enc, b_enc.reshape(1, L), b_dec.reshape(1, D))


def _sel_body(pre_ref, rows_ref, vals_ref):
    i = pl.program_id(0)
    acts = pre_ref[...]
    nch = acts.shape[1] // CHUNK
    cmax = jnp.max(acts.reshape(TM2, nch, CHUNK), axis=-1)
    iota_c = lax.broadcasted_iota(jnp.int32, (TM2, nch), 1)
    iota_k = lax.broadcasted_iota(jnp.int32, (TM2, NCAND), 1)

    def body(r, carry):
        cm, vals, ids = carry
        m = jnp.max(cm, axis=1, keepdims=True)
        idx = jnp.min(jnp.where(cm == m, iota_c, nch), axis=1, keepdims=True)
        sel = iota_k == r
        vals = jnp.where(sel, m, vals)
        ids = jnp.where(sel, idx, ids)
        cm = jnp.where(iota_c == idx, -1.0, cm)
        return cm, vals, ids

    vals0 = jnp.zeros((TM2, NCAND), jnp.float32)
    ids0 = jnp.zeros((TM2, NCAND), jnp.int32)
    _, vals, ids = lax.fori_loop(0, NCAND, body, (cmax, vals0, ids0))
    tok = i * TM2 + lax.broadcasted_iota(jnp.int32, (TM2, NCAND), 0)
    rows_ref[...] = tok * nch + ids
    vals_ref[...] = vals


def _select(pre):
    N, L = pre.shape
    rows, vals = pl.pallas_call(
        _sel_body,
        grid=(N // TM2,),
        in_specs=[pl.BlockSpec((TM2, L), lambda i: (i, 0))],
        out_specs=[
            pl.BlockSpec((TM2, NCAND), lambda i: (i, 0)),
            pl.BlockSpec((TM2, NCAND), lambda i: (i, 0)),
        ],
        out_shape=[
            jax.ShapeDtypeStruct((N, NCAND), jnp.int32),
            jax.ShapeDtypeStruct((N, NCAND), jnp.float32),
        ],
    )(pre)
    return rows, vals


def _sc_topk_decode(pre, cand_rows, cand_vals, W_dec, b_dec):
    N, L = pre.shape
    D = W_dec.shape[1]
    nch = L // CHUNK
    pre_rows = pre.reshape(N * nch, CHUNK)
    info = plsc.get_sparse_core_info()
    nw = info.num_cores * info.num_subcores
    tpw = N // nw
    nv = NCAND // 16
    gk = TOPK // 16
    nseg = D // 256
    mesh = plsc.VectorSubcoreMesh(core_axis_name="c", subcore_axis_name="s")

    @functools.partial(
        pl.kernel,
        out_type=[
            jax.ShapeDtypeStruct((N, D), jnp.float32),
            jax.ShapeDtypeStruct((N, TOPK), jnp.float32),
            jax.ShapeDtypeStruct((N, TOPK), jnp.int32),
        ],
        mesh=mesh,
        compiler_params=pltpu.CompilerParams(needs_layout_passes=False),
        scratch_types=[
            pltpu.VMEM((NCAND,), jnp.int32),
            pltpu.VMEM((NCAND,), jnp.float32),
            pltpu.VMEM((NCAND, CHUNK), jnp.float32),
            pltpu.VMEM((TOPK,), jnp.float32),
            pltpu.VMEM((TOPK,), jnp.int32),
            pltpu.VMEM((TOPK * 16,), jnp.float32),
            pltpu.VMEM((2, 16, D), jnp.float32),
            pltpu.VMEM((D,), jnp.float32),
            pltpu.VMEM((D,), jnp.float32),
            pltpu.SemaphoreType.DMA,
            pltpu.SemaphoreType.DMA,
        ],
    )
    def body(pre_hbm, crows_hbm, cvals_hbm, wdec_hbm, bdec_hbm,
             sae_hbm, tact_hbm, tidx_hbm,
             crow, cvals, cand, out_vals, out_idx, out_bcast, wrows, acc, bdec,
             sem0, sem1):
        wid = lax.axis_index("s") * info.num_cores + lax.axis_index("c")
        t0 = wid * tpw
        pltpu.sync_copy(bdec_hbm, bdec)
        lanes = lax.iota(jnp.int32, 16)
        big = jnp.full((16,), 2**30, jnp.int32)

        def token_body(tok, _):
            t = t0 + tok
            pltpu.sync_copy(crows_hbm.at[t], crow)
            pltpu.sync_copy(cvals_hbm.at[t], cvals)
            pltpu.async_copy(pre_hbm.at[crow], cand, sem0).wait()
            cmax0 = [cvals[pl.ds(16 * i, 16)] for i in range(nv)]
            cid = [crow[pl.ds(16 * i, 16)] - t * nch for i in range(nv)]
            cpos = [lanes + 16 * i for i in range(nv)]

            def extract(r, cmax):
                cmax = list(cmax)
                m = cmax[0]
                for i in range(1, nv):
                    m = jnp.maximum(m, cmax[i])
                m_v = jnp.full((16,), jnp.max(m), jnp.float32)
                k = big
                for i in range(nv):
                    k = jnp.minimum(k, jnp.where(cmax[i] == m_v, cid[i], big))
                cmin_v = jnp.full((16,), jnp.min(k), jnp.int32)
                p = big
                for i in range(nv):
                    p = jnp.minimum(p, jnp.where(cid[i] == cmin_v, cpos[i], big))
                p_v = jnp.full((16,), jnp.min(p), jnp.int32)
                lane_key = big
                vs = []
                for j in range(CHUNK // 16):
                    v = plsc.load_gather(cand, [p_v, lanes + 16 * j])
                    vs.append(v)
                    lane_key = jnp.minimum(
                        lane_key, jnp.where(v == m_v, lanes + 16 * j, big))
                lane_v = jnp.full((16,), jnp.min(lane_key), jnp.int32)
                newmax = jnp.full((16,), -1.0, jnp.float32)
                for j in range(CHUNK // 16):
                    v2 = jnp.where(lanes + 16 * j == lane_v, -1.0, vs[j])
                    newmax = jnp.maximum(newmax, v2)
                one = lanes == 0
                r_v = jnp.full((16,), r, jnp.int32)
                plsc.store_scatter(out_vals, [r_v], m_v, mask=one)
                plsc.store_scatter(out_bcast, [r_v * 16 + lanes], m_v)
                plsc.store_scatter(out_idx, [r_v], cmin_v * CHUNK + lane_v,
                                   mask=one)
                plsc.store_scatter(cand, [p_v, lane_v],
                                   jnp.full((16,), -1.0, jnp.float32), mask=one)
                nm_v = jnp.full((16,), jnp.max(newmax), jnp.float32)
                return tuple(
                    jnp.where(cpos[i] == p_v, nm_v, cmax[i]) for i in range(nv))

            lax.fori_loop(0, TOPK, extract, tuple(cmax0))
            pltpu.sync_copy(out_vals, tact_hbm.at[t])
            pltpu.sync_copy(out_idx, tidx_hbm.at[t])

            copies = [pltpu.async_copy(
                wdec_hbm.at[out_idx[pl.ds(0, 16)]], wrows.at[0], sem0)]
            for g in range(gk):
                if g + 1 < gk:
                    copies.append(pltpu.async_copy(
                        wdec_hbm.at[out_idx[pl.ds(16 * (g + 1), 16)]],
                        wrows.at[(g + 1) % 2],
                        sem1 if (g + 1) % 2 else sem0))
                copies[g].wait()
                a = [out_bcast[pl.ds((16 * g + r_) * 16, 16)]
                     for r_ in range(16)]
                src = bdec if g == 0 else acc
                g_v = jnp.full((16,), g % 2, jnp.int32)
                r_vs = [jnp.full((16,), r_, jnp.int32) for r_ in range(16)]

                def seg_body(s, _, g_v=g_v, r_vs=r_vs, a=a, src=src):
                    base = s * 256
                    cols = [lanes + (base + 16 * j) for j in range(16)]
                    regs = [src[pl.ds(base + 16 * j, 16)] for j in range(16)]
                    for r_ in range(16):
                        for j in range(16):
                            w = plsc.load_gather(wrows, [g_v, r_vs[r_], cols[j]])
                            regs[j] = regs[j] + a[r_] * w
                    for j in range(16):
                        acc[pl.ds(base + 16 * j, 16)] = regs[j]
                    return 0

                lax.fori_loop(0, nseg, seg_body, 0)
            pltpu.sync_copy(acc, sae_hbm.at[t])
            return 0

        lax.fori_loop(0, tpw, token_body, 0)

    return body(pre_rows, cand_rows, cand_vals, W_dec, b_dec)


def _fvu_body(x_ref, sae_ref, fvu_ref):
    x = x_ref[...]
    e = x - sae_ref[...]
    l2 = jnp.sum(e * e)
    colsum = jnp.sum(x, axis=0)
    tv = jnp.sum(x * x) - jnp.sum(colsum * colsum) / x.shape[0]
    fvu_ref[0, 0] = l2 / tv


def _fvu(x, sae):
    N, D = x.shape
    out = pl.pallas_call(
        _fvu_body,
        in_specs=[
            pl.BlockSpec((N, D), lambda: (0, 0)),
            pl.BlockSpec((N, D), lambda: (0, 0)),
        ],
        out_specs=pl.BlockSpec(memory_space=pltpu.SMEM),
        out_shape=jax.ShapeDtypeStruct((1, 1), jnp.float32),
    )(x, sae)
    return out.reshape(())


def kernel(x, W_enc, b_enc, W_dec, b_dec):
    pre = _encode(x, W_enc, b_enc, b_dec)
    cand_rows, cand_vals = _select(pre)
    sae_out, top_acts, top_indices = _sc_topk_decode(
        pre, cand_rows, cand_vals, W_dec, b_dec)
    fvu = _fvu(x, sae_out)
    return sae_out, top_acts, top_indices, fvu


# speedup vs baseline: 7.5271x; 1.2161x over previous
import functools

import jax
import jax.numpy as jnp
from jax import lax
from jax.experimental import pallas as pl
from jax.experimental.pallas import tpu as pltpu
from jax.experimental.pallas import tpu_sc as plsc

TOPK = 64
CHUNK = 128
NCAND = 80
TM = 256
TL = 1024
TM2 = 128


def _enc_body(x_ref, wenc_ref, benc_ref, bdec_ref, pre_ref):
    xc = x_ref[...] - bdec_ref[...]
    acc = jnp.dot(xc, wenc_ref[...], preferred_element_type=jnp.float32)
    pre_ref[...] = jnp.maximum(acc + benc_ref[...], 0.0)


def _encode(x, W_enc, b_enc, b_dec):
    N, D = x.shape
    L = W_enc.shape[1]
    return pl.pallas_call(
        _enc_body,
        grid=(N // TM, L // TL),
        in_specs=[
            pl.BlockSpec((TM, D), lambda i, j: (i, 0)),
            pl.BlockSpec((D, TL), lambda i, j: (0, j)),
            pl.BlockSpec((1, TL), lambda i, j: (0, j)),
            pl.BlockSpec((1, D), lambda i, j: (0, 0)),
        ],
        out_specs=pl.BlockSpec((TM, TL), lambda i, j: (i, j)),
        out_shape=jax.ShapeDtypeStruct((N, L), jnp.float32),
    )(x, W_enc, b_enc.reshape(1, L), b_dec.reshape(1, D))


def _sel_body(pre_ref, rows_ref, vals_ref):
    i = pl.program_id(0)
    acts = pre_ref[...]
    nch = acts.shape[1] // CHUNK
    cmax = jnp.max(acts.reshape(TM2, nch, CHUNK), axis=-1)
    iota_c = lax.broadcasted_iota(jnp.int32, (TM2, nch), 1)
    iota_k = lax.broadcasted_iota(jnp.int32, (TM2, NCAND), 1)

    def body(r, carry):
        cm, vals, ids = carry
        m = jnp.max(cm, axis=1, keepdims=True)
        idx = jnp.min(jnp.where(cm == m, iota_c, nch), axis=1, keepdims=True)
        sel = iota_k == r
        vals = jnp.where(sel, m, vals)
        ids = jnp.where(sel, idx, ids)
        cm = jnp.where(iota_c == idx, -1.0, cm)
        return cm, vals, ids

    vals0 = jnp.zeros((TM2, NCAND), jnp.float32)
    ids0 = jnp.zeros((TM2, NCAND), jnp.int32)
    _, vals, ids = lax.fori_loop(0, NCAND, body, (cmax, vals0, ids0))
    tok = i * TM2 + lax.broadcasted_iota(jnp.int32, (TM2, NCAND), 0)
    rows_ref[...] = tok * nch + ids
    vals_ref[...] = vals


def _select(pre):
    N, L = pre.shape
    rows, vals = pl.pallas_call(
        _sel_body,
        grid=(N // TM2,),
        in_specs=[pl.BlockSpec((TM2, L), lambda i: (i, 0))],
        out_specs=[
            pl.BlockSpec((TM2, NCAND), lambda i: (i, 0)),
            pl.BlockSpec((TM2, NCAND), lambda i: (i, 0)),
        ],
        out_shape=[
            jax.ShapeDtypeStruct((N, NCAND), jnp.int32),
            jax.ShapeDtypeStruct((N, NCAND), jnp.float32),
        ],
    )(pre)
    return rows, vals


def _sc_topk_decode(pre, cand_rows, cand_vals, W_dec, b_dec):
    N, L = pre.shape
    D = W_dec.shape[1]
    nch = L // CHUNK
    pre_rows = pre.reshape(N * nch, CHUNK)
    info = plsc.get_sparse_core_info()
    nw = info.num_cores * info.num_subcores
    tpw = N // nw
    nv = NCAND // 16
    gk = TOPK // 16
    nseg = D // 256
    mesh = plsc.VectorSubcoreMesh(core_axis_name="c", subcore_axis_name="s")

    @functools.partial(
        pl.kernel,
        out_type=[
            jax.ShapeDtypeStruct((N, D), jnp.float32),
            jax.ShapeDtypeStruct((N, TOPK), jnp.float32),
            jax.ShapeDtypeStruct((N, TOPK), jnp.int32),
        ],
        mesh=mesh,
        compiler_params=pltpu.CompilerParams(needs_layout_passes=False),
        scratch_types=[
            pltpu.VMEM((NCAND,), jnp.int32),
            pltpu.VMEM((NCAND,), jnp.float32),
            pltpu.VMEM((NCAND, CHUNK), jnp.float32),
            pltpu.VMEM((TOPK,), jnp.float32),
            pltpu.VMEM((TOPK,), jnp.int32),
            pltpu.VMEM((TOPK * 16,), jnp.float32),
            pltpu.VMEM((2, 16, D), jnp.float32),
            pltpu.VMEM((D,), jnp.float32),
            pltpu.VMEM((D,), jnp.float32),
            pltpu.SemaphoreType.DMA,
            pltpu.SemaphoreType.DMA,
        ],
    )
    def body(pre_hbm, crows_hbm, cvals_hbm, wdec_hbm, bdec_hbm,
             sae_hbm, tact_hbm, tidx_hbm,
             crow, cvals, cand, out_vals, out_idx, out_bcast, wrows, acc, bdec,
             sem0, sem1):
        wid = lax.axis_index("s") * info.num_cores + lax.axis_index("c")
        t0 = wid * tpw
        pltpu.sync_copy(bdec_hbm, bdec)
        lanes = lax.iota(jnp.int32, 16)
        big = jnp.full((16,), 2**30, jnp.int32)

        def token_body(tok, _):
            t = t0 + tok
            pltpu.sync_copy(crows_hbm.at[t], crow)
            pltpu.sync_copy(cvals_hbm.at[t], cvals)
            pltpu.async_copy(pre_hbm.at[crow], cand, sem0).wait()
            cmax0 = [cvals[pl.ds(16 * i, 16)] for i in range(nv)]
            cid = [crow[pl.ds(16 * i, 16)] - t * nch for i in range(nv)]
            cpos = [lanes + 16 * i for i in range(nv)]

            def extract(r, cmax):
                cmax = list(cmax)
                m = cmax[0]
                for i in range(1, nv):
                    m = jnp.maximum(m, cmax[i])
                m_v = jnp.full((16,), jnp.max(m), jnp.float32)
                k = big
                for i in range(nv):
                    k = jnp.minimum(k, jnp.where(cmax[i] == m_v, cid[i], big))
                cmin_v = jnp.full((16,), jnp.min(k), jnp.int32)
                p = big
                for i in range(nv):
                    p = jnp.minimum(p, jnp.where(cid[i] == cmin_v, cpos[i], big))
                p_v = jnp.full((16,), jnp.min(p), jnp.int32)
                lane_key = big
                vs = []
                for j in range(CHUNK // 16):
                    v = plsc.load_gather(cand, [p_v, lanes + 16 * j])
                    vs.append(v)
                    lane_key = jnp.minimum(
                        lane_key, jnp.where(v == m_v, lanes + 16 * j, big))
                lane_v = jnp.full((16,), jnp.min(lane_key), jnp.int32)
                newmax = jnp.full((16,), -1.0, jnp.float32)
                for j in range(CHUNK // 16):
                    v2 = jnp.where(lanes + 16 * j == lane_v, -1.0, vs[j])
                    newmax = jnp.maximum(newmax, v2)
                one = lanes == 0
                r_v = jnp.full((16,), r, jnp.int32)
                plsc.store_scatter(out_vals, [r_v], m_v, mask=one)
                plsc.store_scatter(out_bcast, [r_v * 16 + lanes], m_v)
                plsc.store_scatter(out_idx, [r_v], cmin_v * CHUNK + lane_v,
                                   mask=one)
                plsc.store_scatter(cand, [p_v, lane_v],
                                   jnp.full((16,), -1.0, jnp.float32), mask=one)
                nm_v = jnp.full((16,), jnp.max(newmax), jnp.float32)
                return tuple(
                    jnp.where(cpos[i] == p_v, nm_v, cmax[i]) for i in range(nv))

            lax.fori_loop(0, TOPK, extract, tuple(cmax0))
            pltpu.sync_copy(out_vals, tact_hbm.at[t])
            pltpu.sync_copy(out_idx, tidx_hbm.at[t])

            copies = [pltpu.async_copy(
                wdec_hbm.at[out_idx[pl.ds(0, 16)]], wrows.at[0], sem0)]
            for g in range(gk):
                if g + 1 < gk:
                    copies.append(pltpu.async_copy(
                        wdec_hbm.at[out_idx[pl.ds(16 * (g + 1), 16)]],
                        wrows.at[(g + 1) % 2],
                        sem1 if (g + 1) % 2 else sem0))
                copies[g].wait()
                a = [out_bcast[pl.ds((16 * g + r_) * 16, 16)]
                     for r_ in range(16)]
                src = bdec if g == 0 else acc
                g_v = jnp.full((16,), g % 2, jnp.int32)
                r_vs = [jnp.full((16,), r_, jnp.int32) for r_ in range(16)]

                def seg_body(s, _, g_v=g_v, r_vs=r_vs, a=a, src=src):
                    base = s * 256
                    cols = [lanes + (base + 16 * j) for j in range(16)]
                    regs = [src[pl.ds(base + 16 * j, 16)] for j in range(16)]
                    for r_ in range(16):
                        for j in range(16):
                            w = plsc.load_gather(wrows, [g_v, r_vs[r_], cols[j]])
                            regs[j] = regs[j] + a[r_] * w
                    for j in range(16):
                        acc[pl.ds(base + 16 * j, 16)] = regs[j]
                    return 0

                lax.fori_loop(0, nseg, seg_body, 0)
            pltpu.sync_copy(acc, sae_hbm.at[t])
            return 0

        lax.fori_loop(0, tpw, token_body, 0)

    return body(pre_rows, cand_rows, cand_vals, W_dec, b_dec)


def _fvu_body(x_ref, sae_ref, fvu_ref):
    x = x_ref[...]
    e = x - sae_ref[...]
    l2 = jnp.sum(e * e)
    colsum = jnp.sum(x, axis=0)
    tv = jnp.sum(x * x) - jnp.sum(colsum * colsum) / x.shape[0]
    fvu_ref[0, 0] = l2 / tv


def _fvu(x, sae):
    N, D = x.shape
    out = pl.pallas_call(
        _fvu_body,
        in_specs=[
            pl.BlockSpec((N, D), lambda: (0, 0)),
            pl.BlockSpec((N, D), lambda: (0, 0)),
        ],
        out_specs=pl.BlockSpec(memory_space=pltpu.SMEM),
        out_shape=jax.ShapeDtypeStruct((1, 1), jnp.float32),
    )(x, sae)
    return out.reshape(())


def kernel(x, W_enc, b_enc, W_dec, b_dec):
    N = x.shape[0]
    h = N // 2
    outs = []
    for xh in (x[:h], x[h:]):
        pre = _encode(xh, W_enc, b_enc, b_dec)
        cand_rows, cand_vals = _select(pre)
        outs.append(_sc_topk_decode(pre, cand_rows, cand_vals, W_dec, b_dec))
    sae_out = jnp.concatenate([outs[0][0], outs[1][0]], axis=0)
    top_acts = jnp.concatenate([outs[0][1], outs[1][1]], axis=0)
    top_indices = jnp.concatenate([outs[0][2], outs[1][2]], axis=0)
    fvu = _fvu(x, sae_out)
    return sae_out, top_acts, top_indices, fvu


# speedup vs baseline: 8.3775x; 1.1130x over previous
import functools

import jax
import jax.numpy as jnp
from jax import lax
from jax.experimental import pallas as pl
from jax.experimental.pallas import tpu as pltpu
from jax.experimental.pallas import tpu_sc as plsc

TOPK = 64
CHUNK = 128
NCAND = 80
TM = 256
TL = 1024
TM2 = 128


def _enc_body(x_ref, wenc_ref, benc_ref, bdec_ref, pre_ref):
    xc = x_ref[...] - bdec_ref[...]
    acc = jnp.dot(xc, wenc_ref[...], preferred_element_type=jnp.float32)
    pre_ref[...] = jnp.maximum(acc + benc_ref[...], 0.0)


def _encode(x, W_enc, b_enc, b_dec):
    N, D = x.shape
    L = W_enc.shape[1]
    return pl.pallas_call(
        _enc_body,
        grid=(N // TM, L // TL),
        in_specs=[
            pl.BlockSpec((TM, D), lambda i, j: (i, 0)),
            pl.BlockSpec((D, TL), lambda i, j: (0, j)),
            pl.BlockSpec((1, TL), lambda i, j: (0, j)),
            pl.BlockSpec((1, D), lambda i, j: (0, 0)),
        ],
        out_specs=pl.BlockSpec((TM, TL), lambda i, j: (i, j)),
        out_shape=jax.ShapeDtypeStruct((N, L), jnp.float32),
    )(x, W_enc, b_enc.reshape(1, L), b_dec.reshape(1, D))


def _sel_body(pre_ref, rows_ref, vals_ref):
    i = pl.program_id(0)
    acts = pre_ref[...]
    nch = acts.shape[1] // CHUNK
    cmax = jnp.max(acts.reshape(TM2, nch, CHUNK), axis=-1)
    iota_c = lax.broadcasted_iota(jnp.int32, (TM2, nch), 1)
    iota_k = lax.broadcasted_iota(jnp.int32, (TM2, NCAND), 1)

    def body(r, carry):
        cm, vals, ids = carry
        m = jnp.max(cm, axis=1, keepdims=True)
        idx = jnp.min(jnp.where(cm == m, iota_c, nch), axis=1, keepdims=True)
        sel = iota_k == r
        vals = jnp.where(sel, m, vals)
        ids = jnp.where(sel, idx, ids)
        cm = jnp.where(iota_c == idx, -1.0, cm)
        return cm, vals, ids

    vals0 = jnp.zeros((TM2, NCAND), jnp.float32)
    ids0 = jnp.zeros((TM2, NCAND), jnp.int32)
    _, vals, ids = lax.fori_loop(0, NCAND, body, (cmax, vals0, ids0))
    tok = i * TM2 + lax.broadcasted_iota(jnp.int32, (TM2, NCAND), 0)
    rows_ref[...] = tok * nch + ids
    vals_ref[...] = vals


def _select(pre):
    N, L = pre.shape
    rows, vals = pl.pallas_call(
        _sel_body,
        grid=(N // TM2,),
        in_specs=[pl.BlockSpec((TM2, L), lambda i: (i, 0))],
        out_specs=[
            pl.BlockSpec((TM2, NCAND), lambda i: (i, 0)),
            pl.BlockSpec((TM2, NCAND), lambda i: (i, 0)),
        ],
        out_shape=[
            jax.ShapeDtypeStruct((N, NCAND), jnp.int32),
            jax.ShapeDtypeStruct((N, NCAND), jnp.float32),
        ],
    )(pre)
    return rows, vals


def _sc_topk_decode(pre, cand_rows, cand_vals, W_dec, b_dec):
    N, L = pre.shape
    D = W_dec.shape[1]
    nch = L // CHUNK
    pre_rows = pre.reshape(N * nch, CHUNK)
    info = plsc.get_sparse_core_info()
    nw = info.num_cores * info.num_subcores
    tpw = N // nw
    nv = NCAND // 16
    gk = TOPK // 16
    nseg = D // 256
    mesh = plsc.VectorSubcoreMesh(core_axis_name="c", subcore_axis_name="s")

    @functools.partial(
        pl.kernel,
        out_type=[
            jax.ShapeDtypeStruct((N, D), jnp.float32),
            jax.ShapeDtypeStruct((N, TOPK), jnp.float32),
            jax.ShapeDtypeStruct((N, TOPK), jnp.int32),
        ],
        mesh=mesh,
        compiler_params=pltpu.CompilerParams(needs_layout_passes=False),
        scratch_types=[
            pltpu.VMEM((NCAND,), jnp.int32),
            pltpu.VMEM((NCAND,), jnp.float32),
            pltpu.VMEM((NCAND, CHUNK), jnp.float32),
            pltpu.VMEM((TOPK,), jnp.float32),
            pltpu.VMEM((TOPK,), jnp.int32),
            pltpu.VMEM((TOPK * 16,), jnp.float32),
            pltpu.VMEM((2, 16, D), jnp.float32),
            pltpu.VMEM((D,), jnp.float32),
            pltpu.VMEM((D,), jnp.float32),
            pltpu.SemaphoreType.DMA,
            pltpu.SemaphoreType.DMA,
        ],
    )
    def body(pre_hbm, crows_hbm, cvals_hbm, wdec_hbm, bdec_hbm,
             sae_hbm, tact_hbm, tidx_hbm,
             crow, cvals, cand, out_vals, out_idx, out_bcast, wrows, acc, bdec,
             sem0, sem1):
        wid = lax.axis_index("s") * info.num_cores + lax.axis_index("c")
        t0 = wid * tpw
        pltpu.sync_copy(bdec_hbm, bdec)
        lanes = lax.iota(jnp.int32, 16)
        big = jnp.full((16,), 2**30, jnp.int32)

        def token_body(tok, _):
            t = t0 + tok
            pltpu.sync_copy(crows_hbm.at[t], crow)
            pltpu.sync_copy(cvals_hbm.at[t], cvals)
            pltpu.async_copy(pre_hbm.at[crow], cand, sem0).wait()
            cmax0 = [cvals[pl.ds(16 * i, 16)] for i in range(nv)]
            cid = [crow[pl.ds(16 * i, 16)] - t * nch for i in range(nv)]
            cpos = [lanes + 16 * i for i in range(nv)]

            def extract(r, cmax):
                cmax = list(cmax)
                m = cmax[0]
                for i in range(1, nv):
                    m = jnp.maximum(m, cmax[i])
                m_v = jnp.full((16,), jnp.max(m), jnp.float32)
                k = big
                for i in range(nv):
                    k = jnp.minimum(k, jnp.where(cmax[i] == m_v, cid[i], big))
                cmin_v = jnp.full((16,), jnp.min(k), jnp.int32)
                p = big
                for i in range(nv):
                    p = jnp.minimum(p, jnp.where(cid[i] == cmin_v, cpos[i], big))
                p_v = jnp.full((16,), jnp.min(p), jnp.int32)
                lane_key = big
                vs = []
                for j in range(CHUNK // 16):
                    v = plsc.load_gather(cand, [p_v, lanes + 16 * j])
                    vs.append(v)
                    lane_key = jnp.minimum(
                        lane_key, jnp.where(v == m_v, lanes + 16 * j, big))
                lane_v = jnp.full((16,), jnp.min(lane_key), jnp.int32)
                newmax = jnp.full((16,), -1.0, jnp.float32)
                for j in range(CHUNK // 16):
                    v2 = jnp.where(lanes + 16 * j == lane_v, -1.0, vs[j])
                    newmax = jnp.maximum(newmax, v2)
                one = lanes == 0
                r_v = jnp.full((16,), r, jnp.int32)
                plsc.store_scatter(out_vals, [r_v], m_v, mask=one)
                plsc.store_scatter(out_bcast, [r_v * 16 + lanes], m_v)
                plsc.store_scatter(out_idx, [r_v], cmin_v * CHUNK + lane_v,
                                   mask=one)
                plsc.store_scatter(cand, [p_v, lane_v],
                                   jnp.full((16,), -1.0, jnp.float32), mask=one)
                nm_v = jnp.full((16,), jnp.max(newmax), jnp.float32)
                return tuple(
                    jnp.where(cpos[i] == p_v, nm_v, cmax[i]) for i in range(nv))

            lax.fori_loop(0, TOPK, extract, tuple(cmax0))
            pltpu.sync_copy(out_vals, tact_hbm.at[t])
            pltpu.sync_copy(out_idx, tidx_hbm.at[t])

            copies = [pltpu.async_copy(
                wdec_hbm.at[out_idx[pl.ds(0, 16)]], wrows.at[0], sem0)]
            for g in range(gk):
                if g + 1 < gk:
                    copies.append(pltpu.async_copy(
                        wdec_hbm.at[out_idx[pl.ds(16 * (g + 1), 16)]],
                        wrows.at[(g + 1) % 2],
                        sem1 if (g + 1) % 2 else sem0))
                copies[g].wait()
                a = [out_bcast[pl.ds((16 * g + r_) * 16, 16)]
                     for r_ in range(16)]
                src = bdec if g == 0 else acc
                g_v = jnp.full((16,), g % 2, jnp.int32)
                r_vs = [jnp.full((16,), r_, jnp.int32) for r_ in range(16)]

                def seg_body(s, _, g_v=g_v, r_vs=r_vs, a=a, src=src):
                    base = s * 256
                    cols = [lanes + (base + 16 * j) for j in range(16)]
                    regs = [src[pl.ds(base + 16 * j, 16)] for j in range(16)]
                    for r_ in range(16):
                        for j in range(16):
                            w = plsc.load_gather(wrows, [g_v, r_vs[r_], cols[j]])
                            regs[j] = regs[j] + a[r_] * w
                    for j in range(16):
                        acc[pl.ds(base + 16 * j, 16)] = regs[j]
                    return 0

                lax.fori_loop(0, nseg, seg_body, 0)
            pltpu.sync_copy(acc, sae_hbm.at[t])
            return 0

        lax.fori_loop(0, tpw, token_body, 0)

    return body(pre_rows, cand_rows, cand_vals, W_dec, b_dec)


def _fvu_body(x_ref, sae_ref, fvu_ref):
    x = x_ref[...]
    e = x - sae_ref[...]
    l2 = jnp.sum(e * e)
    colsum = jnp.sum(x, axis=0)
    tv = jnp.sum(x * x) - jnp.sum(colsum * colsum) / x.shape[0]
    fvu_ref[0, 0] = l2 / tv


def _fvu(x, sae):
    N, D = x.shape
    out = pl.pallas_call(
        _fvu_body,
        in_specs=[
            pl.BlockSpec((N, D), lambda: (0, 0)),
            pl.BlockSpec((N, D), lambda: (0, 0)),
        ],
        out_specs=pl.BlockSpec(memory_space=pltpu.SMEM),
        out_shape=jax.ShapeDtypeStruct((1, 1), jnp.float32),
    )(x, sae)
    return out.reshape(())


def kernel(x, W_enc, b_enc, W_dec, b_dec):
    N = x.shape[0]
    h = N // 4
    outs = []
    for xh in (x[:h], x[h:2 * h], x[2 * h:3 * h], x[3 * h:]):
        pre = _encode(xh, W_enc, b_enc, b_dec)
        cand_rows, cand_vals = _select(pre)
        outs.append(_sc_topk_decode(pre, cand_rows, cand_vals, W_dec, b_dec))
    sae_out = jnp.concatenate([o[0] for o in outs], axis=0)
    top_acts = jnp.concatenate([o[1] for o in outs], axis=0)
    top_indices = jnp.concatenate([o[2] for o in outs], axis=0)
    fvu = _fvu(x, sae_out)
    return sae_out, top_acts, top_indices, fvu


# speedup vs baseline: 8.6622x; 1.0340x over previous
import functools

import jax
import jax.numpy as jnp
from jax import lax
from jax.experimental import pallas as pl
from jax.experimental.pallas import tpu as pltpu
from jax.experimental.pallas import tpu_sc as plsc

TOPK = 64
CHUNK = 128
NCAND = 80
TM = 256
TL = 1024
TM2 = 128


def _enc_body(x_ref, wenc_ref, benc_ref, bdec_ref, pre_ref):
    xc = x_ref[...] - bdec_ref[...]
    acc = jnp.dot(xc, wenc_ref[...], preferred_element_type=jnp.float32)
    pre_ref[...] = jnp.maximum(acc + benc_ref[...], 0.0)


def _encode(x, W_enc, b_enc, b_dec):
    N, D = x.shape
    L = W_enc.shape[1]
    return pl.pallas_call(
        _enc_body,
        grid=(N // TM, L // TL),
        in_specs=[
            pl.BlockSpec((TM, D), lambda i, j: (i, 0)),
            pl.BlockSpec((D, TL), lambda i, j: (0, j)),
            pl.BlockSpec((1, TL), lambda i, j: (0, j)),
            pl.BlockSpec((1, D), lambda i, j: (0, 0)),
        ],
        out_specs=pl.BlockSpec((TM, TL), lambda i, j: (i, j)),
        out_shape=jax.ShapeDtypeStruct((N, L), jnp.float32),
    )(x, W_enc, b_enc.reshape(1, L), b_dec.reshape(1, D))


def _sel_body(pre_ref, rows_ref, vals_ref):
    i = pl.program_id(0)
    acts = pre_ref[...]
    nch = acts.shape[1] // CHUNK
    cmax = jnp.max(acts.reshape(TM2, nch, CHUNK), axis=-1)
    iota_c = lax.broadcasted_iota(jnp.int32, (TM2, nch), 1)
    iota_k = lax.broadcasted_iota(jnp.int32, (TM2, NCAND), 1)

    def body(r, carry):
        cm, vals, ids = carry
        m = jnp.max(cm, axis=1, keepdims=True)
        idx = jnp.min(jnp.where(cm == m, iota_c, nch), axis=1, keepdims=True)
        sel = iota_k == r
        vals = jnp.where(sel, m, vals)
        ids = jnp.where(sel, idx, ids)
        cm = jnp.where(iota_c == idx, -1.0, cm)
        return cm, vals, ids

    vals0 = jnp.zeros((TM2, NCAND), jnp.float32)
    ids0 = jnp.zeros((TM2, NCAND), jnp.int32)
    _, vals, ids = lax.fori_loop(0, NCAND, body, (cmax, vals0, ids0))
    tok = i * TM2 + lax.broadcasted_iota(jnp.int32, (TM2, NCAND), 0)
    rows_ref[...] = tok * nch + ids
    vals_ref[...] = vals


def _select(pre):
    N, L = pre.shape
    rows, vals = pl.pallas_call(
        _sel_body,
        grid=(N // TM2,),
        in_specs=[pl.BlockSpec((TM2, L), lambda i: (i, 0))],
        out_specs=[
            pl.BlockSpec((TM2, NCAND), lambda i: (i, 0)),
            pl.BlockSpec((TM2, NCAND), lambda i: (i, 0)),
        ],
        out_shape=[
            jax.ShapeDtypeStruct((N, NCAND), jnp.int32),
            jax.ShapeDtypeStruct((N, NCAND), jnp.float32),
        ],
    )(pre)
    return rows, vals


def _sc_topk_decode(pre, cand_rows, cand_vals, W_dec, b_dec):
    N, L = pre.shape
    D = W_dec.shape[1]
    nch = L // CHUNK
    pre_rows = pre.reshape(N * nch, CHUNK)
    info = plsc.get_sparse_core_info()
    nw = info.num_cores * info.num_subcores
    tpw = N // nw
    nv = NCAND // 16
    gk = TOPK // 16
    nseg = D // 256
    mesh = plsc.VectorSubcoreMesh(core_axis_name="c", subcore_axis_name="s")

    @functools.partial(
        pl.kernel,
        out_type=[
            jax.ShapeDtypeStruct((N, D), jnp.float32),
            jax.ShapeDtypeStruct((N, TOPK), jnp.float32),
            jax.ShapeDtypeStruct((N, TOPK), jnp.int32),
        ],
        mesh=mesh,
        compiler_params=pltpu.CompilerParams(needs_layout_passes=False),
        scratch_types=[
            pltpu.VMEM((NCAND,), jnp.int32),
            pltpu.VMEM((NCAND,), jnp.float32),
            pltpu.VMEM((NCAND, CHUNK), jnp.float32),
            pltpu.VMEM((TOPK,), jnp.float32),
            pltpu.VMEM((TOPK,), jnp.int32),
            pltpu.VMEM((TOPK * 16,), jnp.float32),
            pltpu.VMEM((2, 16, D), jnp.float32),
            pltpu.VMEM((D,), jnp.float32),
            pltpu.VMEM((D,), jnp.float32),
            pltpu.SemaphoreType.DMA,
            pltpu.SemaphoreType.DMA,
        ],
    )
    def body(pre_hbm, crows_hbm, cvals_hbm, wdec_hbm, bdec_hbm,
             sae_hbm, tact_hbm, tidx_hbm,
             crow, cvals, cand, out_vals, out_idx, out_bcast, wrows, acc, bdec,
             sem0, sem1):
        wid = lax.axis_index("s") * info.num_cores + lax.axis_index("c")
        t0 = wid * tpw
        pltpu.sync_copy(bdec_hbm, bdec)
        lanes = lax.iota(jnp.int32, 16)
        big = jnp.full((16,), 2**30, jnp.int32)

        def token_body(tok, _):
            t = t0 + tok
            pltpu.sync_copy(crows_hbm.at[t], crow)
            pltpu.sync_copy(cvals_hbm.at[t], cvals)
            pltpu.async_copy(pre_hbm.at[crow], cand, sem0).wait()
            cmax0 = [cvals[pl.ds(16 * i, 16)] for i in range(nv)]
            cid = [crow[pl.ds(16 * i, 16)] - t * nch for i in range(nv)]
            cpos = [lanes + 16 * i for i in range(nv)]

            def extract(r, cmax):
                cmax = list(cmax)
                m = cmax[0]
                for i in range(1, nv):
                    m = jnp.maximum(m, cmax[i])
                m_v = jnp.full((16,), jnp.max(m), jnp.float32)
                k = big
                for i in range(nv):
                    k = jnp.minimum(k, jnp.where(cmax[i] == m_v, cid[i], big))
                cmin_v = jnp.full((16,), jnp.min(k), jnp.int32)
                p = big
                for i in range(nv):
                    p = jnp.minimum(p, jnp.where(cid[i] == cmin_v, cpos[i], big))
                p_v = jnp.full((16,), jnp.min(p), jnp.int32)
                lane_key = big
                vs = []
                for j in range(CHUNK // 16):
                    v = plsc.load_gather(cand, [p_v, lanes + 16 * j])
                    vs.append(v)
                    lane_key = jnp.minimum(
                        lane_key, jnp.where(v == m_v, lanes + 16 * j, big))
                lane_v = jnp.full((16,), jnp.min(lane_key), jnp.int32)
                newmax = jnp.full((16,), -1.0, jnp.float32)
                for j in range(CHUNK // 16):
                    v2 = jnp.where(lanes + 16 * j == lane_v, -1.0, vs[j])
                    newmax = jnp.maximum(newmax, v2)
                one = lanes == 0
                r_v = jnp.full((16,), r, jnp.int32)
                plsc.store_scatter(out_vals, [r_v], m_v, mask=one)
                plsc.store_scatter(out_bcast, [r_v * 16 + lanes], m_v)
                plsc.store_scatter(out_idx, [r_v], cmin_v * CHUNK + lane_v,
                                   mask=one)
                plsc.store_scatter(cand, [p_v, lane_v],
                                   jnp.full((16,), -1.0, jnp.float32), mask=one)
                nm_v = jnp.full((16,), jnp.max(newmax), jnp.float32)
                return tuple(
                    jnp.where(cpos[i] == p_v, nm_v, cmax[i]) for i in range(nv))

            lax.fori_loop(0, TOPK, extract, tuple(cmax0))
            pltpu.sync_copy(out_vals, tact_hbm.at[t])
            pltpu.sync_copy(out_idx, tidx_hbm.at[t])

            copies = [pltpu.async_copy(
                wdec_hbm.at[out_idx[pl.ds(0, 16)]], wrows.at[0], sem0)]
            for g in range(gk):
                if g + 1 < gk:
                    copies.append(pltpu.async_copy(
                        wdec_hbm.at[out_idx[pl.ds(16 * (g + 1), 16)]],
                        wrows.at[(g + 1) % 2],
                        sem1 if (g + 1) % 2 else sem0))
                copies[g].wait()
                a = [out_bcast[pl.ds((16 * g + r_) * 16, 16)]
                     for r_ in range(16)]
                src = bdec if g == 0 else acc
                g_v = jnp.full((16,), g % 2, jnp.int32)
                r_vs = [jnp.full((16,), r_, jnp.int32) for r_ in range(16)]

                def seg_body(s, _, g_v=g_v, r_vs=r_vs, a=a, src=src):
                    base = s * 256
                    cols = [lanes + (base + 16 * j) for j in range(16)]
                    regs = [src[pl.ds(base + 16 * j, 16)] for j in range(16)]
                    for r_ in range(16):
                        for j in range(16):
                            w = plsc.load_gather(wrows, [g_v, r_vs[r_], cols[j]])
                            regs[j] = regs[j] + a[r_] * w
                    for j in range(16):
                        acc[pl.ds(base + 16 * j, 16)] = regs[j]
                    return 0

                lax.fori_loop(0, nseg, seg_body, 0)
            pltpu.sync_copy(acc, sae_hbm.at[t])
            return 0

        lax.fori_loop(0, tpw, token_body, 0)

    return body(pre_rows, cand_rows, cand_vals, W_dec, b_dec)


def _fvu_body(x_ref, sae_ref, fvu_ref):
    x = x_ref[...]
    e = x - sae_ref[...]
    l2 = jnp.sum(e * e)
    colsum = jnp.sum(x, axis=0)
    tv = jnp.sum(x * x) - jnp.sum(colsum * colsum) / x.shape[0]
    fvu_ref[0, 0] = l2 / tv


def _fvu(x, sae):
    N, D = x.shape
    out = pl.pallas_call(
        _fvu_body,
        in_specs=[
            pl.BlockSpec((N, D), lambda: (0, 0)),
            pl.BlockSpec((N, D), lambda: (0, 0)),
        ],
        out_specs=pl.BlockSpec(memory_space=pltpu.SMEM),
        out_shape=jax.ShapeDtypeStruct((1, 1), jnp.float32),
    )(x, sae)
    return out.reshape(())


def kernel(x, W_enc, b_enc, W_dec, b_dec):
    N = x.shape[0]
    nsplit = 8
    h = N // nsplit
    outs = []
    for c in range(nsplit):
        xh = x[c * h:(c + 1) * h]
        pre = _encode(xh, W_enc, b_enc, b_dec)
        cand_rows, cand_vals = _select(pre)
        outs.append(_sc_topk_decode(pre, cand_rows, cand_vals, W_dec, b_dec))
    sae_out = jnp.concatenate([o[0] for o in outs], axis=0)
    top_acts = jnp.concatenate([o[1] for o in outs], axis=0)
    top_indices = jnp.concatenate([o[2] for o in outs], axis=0)
    fvu = _fvu(x, sae_out)
    return sae_out, top_acts, top_indices, fvu
